# baseline (speedup 1.0000x reference)
import functools

import jax
import jax.numpy as jnp
from jax import lax
from jax.experimental import pallas as pl
from jax.experimental.pallas import tpu as pltpu
from jax.experimental.pallas import tpu_sc as plsc

T = 32768
B = 16
VEC = 64
NF = 1000000
NC = 2
NS = 16
NW = NC * NS
TW = T // NW
PHASES = 8
PT = TW // PHASES
GROUP = 128
RG = 2 * PT // GROUP
WG = PT // GROUP
FIR = 2 * TW // GROUP
IIR = TW // GROUP

HALF = NF // 2
REPACK_BLK = 4000
RB_N = HALF // REPACK_BLK


def _repack_body(x1_ref, x2_ref, o_ref):
    o_ref[...] = jnp.concatenate([x1_ref[...], x2_ref[...]], axis=1)


def _repack(vecs):
    return pl.pallas_call(
        _repack_body,
        grid=(RB_N,),
        in_specs=[
            pl.BlockSpec((REPACK_BLK, VEC), lambda i: (i, 0)),
            pl.BlockSpec((REPACK_BLK, VEC), lambda i: (i + RB_N, 0)),
        ],
        out_specs=pl.BlockSpec((REPACK_BLK, 2 * VEC), lambda i: (i, 0)),
        out_shape=jax.ShapeDtypeStruct((HALF, 2 * VEC), jnp.float32),
    )(vecs, vecs)


def _sc_partials(feat2d, intr2d, divs, segs, vecs2, intr_w, intr_b):
    mesh = plsc.VectorSubcoreMesh(core_axis_name="c", subcore_axis_name="s")

    @functools.partial(
        pl.kernel,
        out_type=jax.ShapeDtypeStruct((NW, B, 16), jnp.float32),
        mesh=mesh,
        scratch_types=[
            pltpu.VMEM((FIR, GROUP), jnp.int32),
            pltpu.VMEM((FIR, GROUP), jnp.int32),
            pltpu.VMEM((2 * TW,), jnp.int32),
            pltpu.VMEM((IIR, GROUP), jnp.int32),
            pltpu.VMEM((TW + 16,), jnp.float32),
            pltpu.VMEM((TW + 16,), jnp.int32),
            pltpu.VMEM((16,), jnp.float32),
            pltpu.VMEM((2, 2 * PT, 2 * VEC), jnp.float32),
            pltpu.VMEM((2, PT + 16), jnp.float32),
            pltpu.VMEM((4, B, 16), jnp.float32),
            pltpu.VMEM((B, 16), jnp.float32),
            pltpu.SemaphoreType.DMA,
            pltpu.SemaphoreType.DMA,
        ],
    )
    def body(feat_hbm, intr_hbm, divs_hbm, segs_hbm, vecs_hbm, w_hbm, b_hbm,
             out_hbm, fidx_v, pidx_v, hoff_v, iidx_v, divs_v, segs_v, b_v,
             rows_v, w_v, acc_v, out_v, sem0, sem1):
        wid = lax.axis_index("c") * NS + lax.axis_index("s")
        sems = (sem0, sem1)

        pltpu.sync_copy(feat_hbm.at[pl.ds(wid * FIR, FIR)], fidx_v)
        pltpu.sync_copy(intr_hbm.at[pl.ds(wid * IIR, IIR)], iidx_v)
        pltpu.sync_copy(divs_hbm.at[pl.ds(wid * TW, TW)], divs_v.at[pl.ds(0, TW)])
        pltpu.sync_copy(segs_hbm.at[pl.ds(wid * TW, TW)], segs_v.at[pl.ds(0, TW)])
        pltpu.sync_copy(b_hbm, b_v)

        halfv = jnp.full((16,), HALF, jnp.int32)
        zerov = jnp.full((16,), 0, jnp.int32)
        sixty4 = jnp.full((16,), 64, jnp.int32)

        def prep(r, _):
            for c in range(8):
                v = fidx_v[r, pl.ds(c * 16, 16)]
                hi = v >= halfv
                pidx_v[r, pl.ds(c * 16, 16)] = v - jnp.where(hi, halfv, zerov)
                st = pl.multiple_of(r * GROUP + c * 16, 16)
                hoff_v[pl.ds(st, 16)] = jnp.where(hi, sixty4, zerov)
            return 0

        lax.fori_loop(0, FIR, prep, 0)

        zero = jnp.zeros((16,), jnp.float32)
        for i in range(4):
            for s in range(B):
                acc_v[i, s, :] = zero

        bvec = b_v[...] * (1.0 / 16.0)

        def fire(p):
            buf = p % 2
            hs = []
            for j in range(RG):
                hs.append(pltpu.async_copy(
                    vecs_hbm.at[pidx_v.at[RG * p + j]],
                    rows_v.at[buf, pl.ds(j * GROUP, GROUP)],
                    sems[buf]))
            hs.append(pltpu.async_copy(
                w_hbm.at[iidx_v.at[p]],
                w_v.at[buf, pl.ds(0, GROUP)],
                sems[buf]))
            return hs

        def compute(p):
            buf = p % 2

            def grp(gi, _):
                base = pl.multiple_of(gi * 16, 16)
                gbase = pl.multiple_of(p * PT + base, 16)
                cv = w_v[buf, pl.ds(base, 16)] / divs_v[pl.ds(gbase, 16)]
                sv = segs_v[pl.ds(gbase, 16)]
                o0 = hoff_v[pl.ds(pl.multiple_of(2 * gbase, 16), 16)]
                o1 = hoff_v[pl.ds(pl.multiple_of(2 * gbase + 16, 16), 16)]
                for k in range(16):
                    ov, kk = (o0, k) if k < 8 else (o1, k - 8)
                    offa = pl.multiple_of(ov[2 * kk], 64)
                    offb = pl.multiple_of(ov[2 * kk + 1], 64)
                    t2 = 2 * (base + k)
                    s = (rows_v[buf, t2, pl.ds(offa, 16)]
                         * rows_v[buf, t2 + 1, pl.ds(offb, 16)]
                         + rows_v[buf, t2, pl.ds(offa + 16, 16)]
                         * rows_v[buf, t2 + 1, pl.ds(offb + 16, 16)])
                    s = s + (rows_v[buf, t2, pl.ds(offa + 32, 16)]
                             * rows_v[buf, t2 + 1, pl.ds(offb + 32, 16)]
                             + rows_v[buf, t2, pl.ds(offa + 48, 16)]
                             * rows_v[buf, t2 + 1, pl.ds(offb + 48, 16)])
                    sg = sv[k]
                    acc_v[k & 3, sg, :] = (acc_v[k & 3, sg, :]
                                           + (s * jnp.full((16,), cv[k], jnp.float32)
                                              + bvec))
                return 0

            lax.fori_loop(0, PT // 16, grp, 0)

        pending = fire(0)
        for p in range(PHASES):
            nxt = fire(p + 1) if p + 1 < PHASES else []
            for h in pending:
                h.wait()
            compute(p)
            pending = nxt

        for s in range(B):
            out_v[s, :] = ((acc_v[0, s, :] + acc_v[1, s, :])
                           + (acc_v[2, s, :] + acc_v[3, s, :]))
        pltpu.sync_copy(out_v, out_hbm.at[wid])

    return body(feat2d, intr2d, divs, segs, vecs2, intr_w, intr_b)


def _sum_body(x_ref, o_ref):
    o_ref[...] = jnp.sum(jnp.sum(x_ref[...], axis=2), axis=0, keepdims=True)


def kernel(intr_idxs, intr_divs, feat_idxs, segment_ids, vecs, intr_W, intr_b):
    feat2d = feat_idxs.reshape(2 * T // GROUP, GROUP)
    intr2d = intr_idxs.reshape(T // GROUP, GROUP)
    vecs2 = _repack(vecs)
    partials = _sc_partials(feat2d, intr2d, intr_divs, segment_ids,
                            vecs2, intr_W.reshape(-1), jnp.tile(intr_b, 16))
    out = pl.pallas_call(
        _sum_body,
        out_shape=jax.ShapeDtypeStruct((1, B), jnp.float32),
    )(partials)
    return out[0]


# speedup vs baseline: 1.0138x; 1.0138x over previous
import functools

import jax
import jax.numpy as jnp
from jax import lax
from jax.experimental import pallas as pl
from jax.experimental.pallas import tpu as pltpu
from jax.experimental.pallas import tpu_sc as plsc

T = 32768
B = 16
VEC = 64
NF = 1000000
NC = 2
NS = 16
NW = NC * NS
TW = T // NW
PHASES = 8
PT = TW // PHASES
GROUP = 128
RG = 2 * PT // GROUP
WG = PT // GROUP
FIR = 2 * TW // GROUP
IIR = TW // GROUP

HALF = NF // 2
REPACK_BLK = 20000
RB_N = HALF // REPACK_BLK


def _repack_body(x1_ref, x2_ref, o_ref):
    o_ref[...] = jnp.concatenate([x1_ref[...], x2_ref[...]], axis=1)


def _repack(vecs):
    return pl.pallas_call(
        _repack_body,
        grid=(RB_N,),
        in_specs=[
            pl.BlockSpec((REPACK_BLK, VEC), lambda i: (i, 0)),
            pl.BlockSpec((REPACK_BLK, VEC), lambda i: (i + RB_N, 0)),
        ],
        out_specs=pl.BlockSpec((REPACK_BLK, 2 * VEC), lambda i: (i, 0)),
        out_shape=jax.ShapeDtypeStruct((HALF, 2 * VEC), jnp.float32),
    )(vecs, vecs)


def _sc_partials(feat2d, intr2d, divs, segs, vecs2, intr_w, intr_b):
    mesh = plsc.VectorSubcoreMesh(core_axis_name="c", subcore_axis_name="s")

    @functools.partial(
        pl.kernel,
        out_type=jax.ShapeDtypeStruct((NW, B, 16), jnp.float32),
        mesh=mesh,
        scratch_types=[
            pltpu.VMEM((FIR, GROUP), jnp.int32),
            pltpu.VMEM((FIR, GROUP), jnp.int32),
            pltpu.VMEM((2 * TW,), jnp.int32),
            pltpu.VMEM((IIR, GROUP), jnp.int32),
            pltpu.VMEM((TW + 16,), jnp.float32),
            pltpu.VMEM((TW + 16,), jnp.int32),
            pltpu.VMEM((16,), jnp.float32),
            pltpu.VMEM((2, 2 * PT, 2 * VEC), jnp.float32),
            pltpu.VMEM((2, PT + 16), jnp.float32),
            pltpu.VMEM((4, B, 16), jnp.float32),
            pltpu.VMEM((B, 16), jnp.float32),
            pltpu.SemaphoreType.DMA,
            pltpu.SemaphoreType.DMA,
        ],
    )
    def body(feat_hbm, intr_hbm, divs_hbm, segs_hbm, vecs_hbm, w_hbm, b_hbm,
             out_hbm, fidx_v, pidx_v, hoff_v, iidx_v, divs_v, segs_v, b_v,
             rows_v, w_v, acc_v, out_v, sem0, sem1):
        wid = lax.axis_index("c") * NS + lax.axis_index("s")
        sems = (sem0, sem1)

        pltpu.sync_copy(feat_hbm.at[pl.ds(wid * FIR, FIR)], fidx_v)
        pltpu.sync_copy(intr_hbm.at[pl.ds(wid * IIR, IIR)], iidx_v)
        pltpu.sync_copy(divs_hbm.at[pl.ds(wid * TW, TW)], divs_v.at[pl.ds(0, TW)])
        pltpu.sync_copy(segs_hbm.at[pl.ds(wid * TW, TW)], segs_v.at[pl.ds(0, TW)])
        pltpu.sync_copy(b_hbm, b_v)

        halfv = jnp.full((16,), HALF, jnp.int32)
        zerov = jnp.full((16,), 0, jnp.int32)
        sixty4 = jnp.full((16,), 64, jnp.int32)

        def prep(r, _):
            for c in range(8):
                v = fidx_v[r, pl.ds(c * 16, 16)]
                hi = v >= halfv
                pidx_v[r, pl.ds(c * 16, 16)] = v - jnp.where(hi, halfv, zerov)
                st = pl.multiple_of(r * GROUP + c * 16, 16)
                hoff_v[pl.ds(st, 16)] = jnp.where(hi, sixty4, zerov)
            return 0

        lax.fori_loop(0, FIR, prep, 0)

        zero = jnp.zeros((16,), jnp.float32)
        for i in range(4):
            for s in range(B):
                acc_v[i, s, :] = zero

        bvec = b_v[...] * (1.0 / 16.0)

        def fire(p):
            buf = p % 2
            hs = []
            for j in range(RG):
                hs.append(pltpu.async_copy(
                    vecs_hbm.at[pidx_v.at[RG * p + j]],
                    rows_v.at[buf, pl.ds(j * GROUP, GROUP)],
                    sems[buf]))
            hs.append(pltpu.async_copy(
                w_hbm.at[iidx_v.at[p]],
                w_v.at[buf, pl.ds(0, GROUP)],
                sems[buf]))
            return hs

        def compute(p):
            buf = p % 2

            def grp(gi, _):
                base = pl.multiple_of(gi * 16, 16)
                gbase = pl.multiple_of(p * PT + base, 16)
                cv = w_v[buf, pl.ds(base, 16)] / divs_v[pl.ds(gbase, 16)]
                sv = segs_v[pl.ds(gbase, 16)]
                o0 = hoff_v[pl.ds(pl.multiple_of(2 * gbase, 16), 16)]
                o1 = hoff_v[pl.ds(pl.multiple_of(2 * gbase + 16, 16), 16)]
                for k in range(16):
                    ov, kk = (o0, k) if k < 8 else (o1, k - 8)
                    offa = pl.multiple_of(ov[2 * kk], 64)
                    offb = pl.multiple_of(ov[2 * kk + 1], 64)
                    t2 = 2 * (base + k)
                    s = (rows_v[buf, t2, pl.ds(offa, 16)]
                         * rows_v[buf, t2 + 1, pl.ds(offb, 16)]
                         + rows_v[buf, t2, pl.ds(offa + 16, 16)]
                         * rows_v[buf, t2 + 1, pl.ds(offb + 16, 16)])
                    s = s + (rows_v[buf, t2, pl.ds(offa + 32, 16)]
                             * rows_v[buf, t2 + 1, pl.ds(offb + 32, 16)]
                             + rows_v[buf, t2, pl.ds(offa + 48, 16)]
                             * rows_v[buf, t2 + 1, pl.ds(offb + 48, 16)])
                    sg = sv[k]
                    acc_v[k & 3, sg, :] = (acc_v[k & 3, sg, :]
                                           + (s * jnp.full((16,), cv[k], jnp.float32)
                                              + bvec))
                return 0

            lax.fori_loop(0, PT // 16, grp, 0)

        pending = fire(0)
        for p in range(PHASES):
            nxt = fire(p + 1) if p + 1 < PHASES else []
            for h in pending:
                h.wait()
            compute(p)
            pending = nxt

        for s in range(B):
            out_v[s, :] = ((acc_v[0, s, :] + acc_v[1, s, :])
                           + (acc_v[2, s, :] + acc_v[3, s, :]))
        pltpu.sync_copy(out_v, out_hbm.at[wid])

    return body(feat2d, intr2d, divs, segs, vecs2, intr_w, intr_b)


def _sum_body(x_ref, o_ref):
    o_ref[...] = jnp.sum(jnp.sum(x_ref[...], axis=2), axis=0, keepdims=True)


def kernel(intr_idxs, intr_divs, feat_idxs, segment_ids, vecs, intr_W, intr_b):
    feat2d = feat_idxs.reshape(2 * T // GROUP, GROUP)
    intr2d = intr_idxs.reshape(T // GROUP, GROUP)
    vecs2 = _repack(vecs)
    partials = _sc_partials(feat2d, intr2d, intr_divs, segment_ids,
                            vecs2, intr_W.reshape(-1), jnp.tile(intr_b, 16))
    out = pl.pallas_call(
        _sum_body,
        out_shape=jax.ShapeDtypeStruct((1, B), jnp.float32),
    )(partials)
    return out[0]


# speedup vs baseline: 1.4844x; 1.4642x over previous
import functools

import jax
import jax.numpy as jnp
from jax import lax
from jax.experimental import pallas as pl
from jax.experimental.pallas import tpu as pltpu
from jax.experimental.pallas import tpu_sc as plsc

T = 32768
B = 16
VEC = 64
NF = 1000000
NC = 2
NS = 16
NW = NC * NS
TW = T // NW
NP = 16
PT = TW // NP
SIDES = 2 * PT
GROUP = 128
FIR = 2 * TW // GROUP
IIR = TW // GROUP


def _sc_partials(feat2d, intr2d, divs, segs, vecs, intr_w, intr_b):
    mesh = plsc.VectorSubcoreMesh(core_axis_name="c", subcore_axis_name="s")

    @functools.partial(
        pl.kernel,
        out_type=jax.ShapeDtypeStruct((NW, B, 16), jnp.float32),
        mesh=mesh,
        scratch_types=[
            pltpu.VMEM((FIR, GROUP), jnp.int32),
            pltpu.VMEM((IIR, GROUP), jnp.int32),
            pltpu.VMEM((TW + 16,), jnp.float32),
            pltpu.VMEM((TW + 16,), jnp.int32),
            pltpu.VMEM((16,), jnp.float32),
            pltpu.VMEM((TW + 16,), jnp.float32),
            pltpu.VMEM((SIDES, VEC), jnp.float32),
            pltpu.VMEM((SIDES, VEC), jnp.float32),
            pltpu.VMEM((4, B, 16), jnp.float32),
            pltpu.VMEM((B, 16), jnp.float32),
            pltpu.SemaphoreType.DMA,
            pltpu.SemaphoreType.DMA,
        ],
    )
    def body(feat_hbm, intr_hbm, divs_hbm, segs_hbm, vecs_hbm, w_hbm, b_hbm,
             out_hbm, fidx_v, iidx_v, divs_v, segs_v, b_v, w_v,
             rows_a, rows_b, acc_v, out_v, sem_a, sem_b):
        wid = lax.axis_index("c") * NS + lax.axis_index("s")

        pltpu.sync_copy(feat_hbm.at[pl.ds(wid * FIR, FIR)], fidx_v)
        pltpu.sync_copy(intr_hbm.at[pl.ds(wid * IIR, IIR)], iidx_v)
        pltpu.sync_copy(divs_hbm.at[pl.ds(wid * TW, TW)], divs_v.at[pl.ds(0, TW)])
        pltpu.sync_copy(segs_hbm.at[pl.ds(wid * TW, TW)], segs_v.at[pl.ds(0, TW)])
        pltpu.sync_copy(b_hbm, b_v)

        for j in range(IIR):
            pltpu.async_copy(w_hbm.at[iidx_v.at[j]],
                             w_v.at[pl.ds(j * GROUP, GROUP)], sem_a).wait()

        zero = jnp.zeros((16,), jnp.float32)
        for i in range(4):
            for s in range(B):
                acc_v[i, s, :] = zero

        bvec = b_v[...] * (1.0 / 16.0)

        def fire(ph, rows, sem):
            for jv in range(8):
                iv = fidx_v[ph, pl.ds(jv * 16, 16)]
                for k in range(16):
                    pltpu.async_copy(vecs_hbm.at[iv[k]],
                                     rows.at[jv * 16 + k], sem)

        def drain(rows, sem):
            pltpu.make_async_copy(vecs_hbm.at[pl.ds(0, SIDES)], rows, sem).wait()

        def compute(ph, rows):
            for g in range(4):
                base = pl.multiple_of(ph * PT + g * 16, 16)
                cv = w_v[pl.ds(base, 16)] / divs_v[pl.ds(base, 16)]
                sv = segs_v[pl.ds(base, 16)]
                for k in range(16):
                    t2 = 2 * (g * 16 + k)
                    s = (rows[t2, pl.ds(0, 16)] * rows[t2 + 1, pl.ds(0, 16)]
                         + rows[t2, pl.ds(16, 16)] * rows[t2 + 1, pl.ds(16, 16)])
                    s = s + (rows[t2, pl.ds(32, 16)] * rows[t2 + 1, pl.ds(32, 16)]
                             + rows[t2, pl.ds(48, 16)] * rows[t2 + 1, pl.ds(48, 16)])
                    sg = sv[k]
                    acc_v[k & 3, sg, :] = (acc_v[k & 3, sg, :]
                                           + (s * jnp.full((16,), cv[k], jnp.float32)
                                              + bvec))

        fire(0, rows_a, sem_a)

        def step(pp, _):
            ph0 = 2 * pp
            fire(ph0 + 1, rows_b, sem_b)
            drain(rows_a, sem_a)
            compute(ph0, rows_a)

            @pl.when(ph0 + 2 < NP)
            def _():
                fire(ph0 + 2, rows_a, sem_a)

            drain(rows_b, sem_b)
            compute(ph0 + 1, rows_b)
            return 0

        lax.fori_loop(0, NP // 2, step, 0)

        for s in range(B):
            out_v[s, :] = ((acc_v[0, s, :] + acc_v[1, s, :])
                           + (acc_v[2, s, :] + acc_v[3, s, :]))
        pltpu.sync_copy(out_v, out_hbm.at[wid])

    return body(feat2d, intr2d, divs, segs, vecs, intr_w, intr_b)


def _sum_body(x_ref, o_ref):
    o_ref[...] = jnp.sum(jnp.sum(x_ref[...], axis=2), axis=0, keepdims=True)


def kernel(intr_idxs, intr_divs, feat_idxs, segment_ids, vecs, intr_W, intr_b):
    feat2d = feat_idxs.reshape(2 * T // GROUP, GROUP)
    intr2d = intr_idxs.reshape(T // GROUP, GROUP)
    partials = _sc_partials(feat2d, intr2d, intr_divs, segment_ids,
                            vecs, intr_W.reshape(-1), jnp.tile(intr_b, 16))
    out = pl.pallas_call(
        _sum_body,
        out_shape=jax.ShapeDtypeStruct((1, B), jnp.float32),
    )(partials)
    return out[0]
